# baseline (speedup 1.0000x reference)
import functools

import jax
import jax.numpy as jnp
from jax import lax
from jax.experimental import pallas as pl
from jax.experimental.pallas import tpu as pltpu
from jax.experimental.pallas import tpu_sc as plsc

N = 10000
E = 160000
BN = 2000

SBLK = 100
NSB = 100
RPT = N // 16
NBUF = 10
LOOK = 5


def _lin0_body(x_ref, w_ref, b_ref, ind_ref, outd_ref, o_ref, rin_ref,
               rout_ref):
    ind = jnp.maximum(ind_ref[:, 0].astype(jnp.float32), 1.0)
    outd = jnp.maximum(outd_ref[:, 0].astype(jnp.float32), 1.0)
    inv = (1.0 / ind)[:, None]
    rin = jax.lax.rsqrt(ind)[:, None]
    rout = jax.lax.rsqrt(outd)[:, None]
    t = (jnp.dot(x_ref[0], w_ref[:64, :], preferred_element_type=jnp.float32)
         + jnp.dot(x_ref[1], w_ref[64:, :], preferred_element_type=jnp.float32))
    t = rout * jnp.maximum(inv * t + b_ref[...][None, :], 0.0)
    o_ref[0] = t[:, :64]
    o_ref[1] = t[:, 64:]
    rin_ref[...] = rin
    rout_ref[...] = rout


def _lin0(a0, w, b, ind, outd):
    dspec = pl.BlockSpec((BN, 16), lambda i: (i, 0))
    sspec = pl.BlockSpec((BN, 1), lambda i: (i, 0))
    return pl.pallas_call(
        _lin0_body,
        grid=(N // BN,),
        in_specs=[
            pl.BlockSpec((2, BN, 64), lambda i: (0, i, 0)),
            pl.BlockSpec((128, 128), lambda i: (0, 0)),
            pl.BlockSpec((128,), lambda i: (0,)),
            dspec,
            dspec,
        ],
        out_specs=[pl.BlockSpec((2, BN, 64), lambda i: (0, i, 0)),
                   sspec, sspec],
        out_shape=[jax.ShapeDtypeStruct((2, N, 64), jnp.float32),
                   jax.ShapeDtypeStruct((N, 1), jnp.float32),
                   jax.ShapeDtypeStruct((N, 1), jnp.float32)],
    )(a0, w, b, ind, outd)


def _lin_body(ngroups, relu, post, *refs):
    x_refs = refs[:ngroups]
    w_ref, b_ref, spre_ref, spost_ref = refs[ngroups:ngroups + 4]
    o_refs = refs[ngroups + 4:]
    t = None
    off = 0
    for x_ref in x_refs:
        fh = x_ref.shape[2]
        for h in range(2):
            part = jnp.dot(x_ref[h].astype(jnp.float32),
                           w_ref[off:off + fh, :],
                           preferred_element_type=jnp.float32)
            t = part if t is None else t + part
            off += fh
    t = spre_ref[...] * t + b_ref[...][None, :]
    if relu:
        t = jnp.maximum(t, 0.0)
    if post:
        t = spost_ref[...] * t
    off = 0
    for o_ref in o_refs:
        hw = o_ref.shape[2]
        o_ref[0] = t[:, off:off + hw].astype(o_ref.dtype)
        o_ref[1] = t[:, off + hw:off + 2 * hw].astype(o_ref.dtype)
        off += 2 * hw


def _linear(x_groups, w, b, s_pre, s_post, relu, out_halves, out_dtypes):
    fo = w.shape[1]
    fi = w.shape[0]
    return pl.pallas_call(
        functools.partial(_lin_body, len(x_groups), relu, s_post is not None),
        grid=(N // BN,),
        in_specs=[pl.BlockSpec((2, BN, g.shape[2]), lambda i: (0, i, 0))
                  for g in x_groups] + [
            pl.BlockSpec((fi, fo), lambda i: (0, 0)),
            pl.BlockSpec((fo,), lambda i: (0,)),
            pl.BlockSpec((BN, 1), lambda i: (i, 0)),
            pl.BlockSpec((BN, 1), lambda i: (i, 0)),
        ],
        out_specs=[pl.BlockSpec((2, BN, hw), lambda i: (0, i, 0))
                   for hw in out_halves],
        out_shape=[jax.ShapeDtypeStruct((2, N, hw), dt)
                   for hw, dt in zip(out_halves, out_dtypes)],
    )(*x_groups, w, b, s_pre, s_pre if s_post is None else s_post)


def _lin2_body(x_ref, w_ref, b_ref, wc_ref, rin_ref, rout_ref, o_ref):
    t = (jnp.dot(x_ref[0].astype(jnp.float32), w_ref[:128, :],
                 preferred_element_type=jnp.float32)
         + jnp.dot(x_ref[1].astype(jnp.float32), w_ref[128:, :],
                   preferred_element_type=jnp.float32))
    t = jnp.maximum(rin_ref[...] * t + b_ref[...][None, :], 0.0)
    y = jnp.dot(rout_ref[...] * t, wc_ref[...],
                preferred_element_type=jnp.float32)
    o_ref[0] = y[:, :32]
    o_ref[1] = y[:, 32:]


def _linear2(x2, w, b, wcat, rin, rout):
    fo = w.shape[1]
    fc = wcat.shape[1]
    return pl.pallas_call(
        _lin2_body,
        grid=(N // BN,),
        in_specs=[
            pl.BlockSpec((2, BN, 128), lambda i: (0, i, 0)),
            pl.BlockSpec((256, fo), lambda i: (0, 0)),
            pl.BlockSpec((fo,), lambda i: (0,)),
            pl.BlockSpec((fo, fc), lambda i: (0, 0)),
            pl.BlockSpec((BN, 1), lambda i: (i, 0)),
            pl.BlockSpec((BN, 1), lambda i: (i, 0)),
        ],
        out_specs=pl.BlockSpec((2, BN, 32), lambda i: (0, i, 0)),
        out_shape=jax.ShapeDtypeStruct((2, N, 32), jnp.float32),
    )(x2, w, b, wcat, rin, rout)


def _head_body(x_ref, rin_ref, bc_ref, noise_ref, mean_ref, ls_ref, z_ref):
    mean = rin_ref[...] * x_ref[0] + bc_ref[...][None, :32]
    log_std = rin_ref[...] * x_ref[1] + bc_ref[...][None, 32:]
    mean_ref[...] = mean
    ls_ref[...] = log_std
    z_ref[...] = mean + noise_ref[...] * jnp.exp(log_std * 0.5)


def _head(x2, rin, bcat, noise):
    o32 = pl.BlockSpec((BN, 32), lambda i: (i, 0))
    return pl.pallas_call(
        _head_body,
        grid=(N // BN,),
        in_specs=[
            pl.BlockSpec((2, BN, 32), lambda i: (0, i, 0)),
            pl.BlockSpec((BN, 1), lambda i: (i, 0)),
            pl.BlockSpec((64,), lambda i: (0,)),
            o32,
        ],
        out_specs=[o32, o32, o32],
        out_shape=[jax.ShapeDtypeStruct((N, 32), jnp.float32)] * 3,
    )(x2, rin, bcat, noise)


def _dec_body(za_ref, zb_ref, o_ref):
    t = jnp.dot(za_ref[...], zb_ref[...].T, preferred_element_type=jnp.float32)
    o_ref[...] = jax.nn.sigmoid(t)


def _decoder(z):
    br, bc = 1000, 1280
    return pl.pallas_call(
        _dec_body,
        grid=(N // br, pl.cdiv(N, bc)),
        in_specs=[
            pl.BlockSpec((br, 32), lambda i, j: (i, 0)),
            pl.BlockSpec((bc, 32), lambda i, j: (j, 0)),
        ],
        out_specs=pl.BlockSpec((br, bc), lambda i, j: (i, j)),
        out_shape=jax.ShapeDtypeStruct((N, N), jnp.float32),
    )(z, z)


def _agg_loop(xc, src_v, dst_v, bufs, agg_sh, gsem, ssem, extra=None):
    @pl.loop(0, NSB, step=NBUF)
    def _(j0):
        for u in range(NBUF):
            j = j0 + u
            if extra is not None:
                extra(j)
            pltpu.make_async_copy(
                xc.at[src_v.at[j]], bufs.at[u], gsem[u]).wait()
            pltpu.make_async_copy(
                bufs.at[u], agg_sh.at[dst_v.at[j]], ssem[u]).start(add=True)
            g2 = j + LOOK
            u2 = (u + LOOK) % NBUF

            @pl.when(g2 < NSB)
            def _():
                @pl.when(g2 >= NBUF)
                def _():
                    pltpu.make_async_copy(
                        bufs.at[u2], agg_sh.at[dst_v.at[g2 - NBUF]],
                        ssem[u2]).wait()

                pltpu.make_async_copy(
                    xc.at[src_v.at[g2]], bufs.at[u2], gsem[u2]).start()

    for u in range(NBUF):
        pltpu.make_async_copy(
            bufs.at[u], agg_sh.at[dst_v.at[NSB - NBUF + u]], ssem[u]).wait()


def _make_agg_sc(FH, dtype):
    mesh = plsc.VectorSubcoreMesh(core_axis_name="c", subcore_axis_name="s")
    scratch = [
        pltpu.VMEM((NSB, SBLK), jnp.int32),
        pltpu.VMEM((NSB, SBLK), jnp.int32),
        pltpu.VMEM((NBUF, SBLK, FH), dtype),
        pltpu.VMEM_SHARED((N, FH), dtype),
    ] + [pltpu.SemaphoreType.DMA] * (2 * NBUF + 1)

    @functools.partial(
        pl.kernel,
        out_type=jax.ShapeDtypeStruct((2, N, FH), dtype),
        mesh=mesh,
        scratch_types=scratch,
        compiler_params=pltpu.CompilerParams(use_tc_tiling_on_sc=False),
    )
    def k(x_hbm, src_hbm, dst_hbm, zer_hbm, out_hbm, src_v, dst_v, bufs,
          agg_sh, *sems):
        gsem = sems[:NBUF]
        ssem = sems[NBUF:2 * NBUF]
        msem = sems[2 * NBUF]
        c = lax.axis_index("c")
        s = lax.axis_index("s")
        cp_src = pltpu.make_async_copy(src_hbm.at[s], src_v, msem)
        cp_dst = pltpu.make_async_copy(dst_hbm.at[s], dst_v, msem)
        cp_zer = pltpu.make_async_copy(
            zer_hbm, agg_sh.at[pl.ds(s * RPT, RPT)], msem)
        cp_src.start()
        cp_dst.start()
        cp_zer.start()
        cp_src.wait()
        cp_dst.wait()
        xc = x_hbm.at[c]
        for g in range(LOOK):
            pltpu.make_async_copy(
                xc.at[src_v.at[g]], bufs.at[g], gsem[g]).start()
        cp_zer.wait()
        plsc.subcore_barrier()
        _agg_loop(xc, src_v, dst_v, bufs, agg_sh, gsem, ssem)
        plsc.subcore_barrier()
        pltpu.async_copy(
            agg_sh.at[pl.ds(s * RPT, RPT)],
            out_hbm.at[c].at[pl.ds(s * RPT, RPT)], msem).wait()

    return k


_AGG32 = _make_agg_sc(32, jnp.float32)
_AGG64 = _make_agg_sc(64, jnp.float32)
_AGG96B = _make_agg_sc(96, jnp.bfloat16)
_AGG128B = _make_agg_sc(128, jnp.bfloat16)


def _make_agg_deg_sc():
    FH = 64
    mesh = plsc.VectorSubcoreMesh(core_axis_name="c", subcore_axis_name="s")
    scratch = [
        pltpu.VMEM((NSB, SBLK), jnp.int32),
        pltpu.VMEM((NSB, SBLK), jnp.int32),
        pltpu.VMEM((NBUF, SBLK, FH), jnp.float32),
        pltpu.VMEM((SBLK, 16), jnp.int16),
        pltpu.VMEM_SHARED((N, FH), jnp.float32),
        pltpu.VMEM_SHARED((N, 16), jnp.int16),
    ] + [pltpu.SemaphoreType.DMA] * (2 * NBUF + 2)

    @functools.partial(
        pl.kernel,
        out_type=[jax.ShapeDtypeStruct((2, N, FH), jnp.float32),
                  jax.ShapeDtypeStruct((N, 16), jnp.int16),
                  jax.ShapeDtypeStruct((N, 16), jnp.int16)],
        mesh=mesh,
        scratch_types=scratch,
        compiler_params=pltpu.CompilerParams(use_tc_tiling_on_sc=False),
    )
    def k(x_hbm, src_hbm, dst_hbm, zer_hbm, z16_hbm, out_hbm, ind_hbm,
          outd_hbm, src_v, dst_v, bufs, ones_v, agg_sh, deg_sh, *sems):
        gsem = sems[:NBUF]
        ssem = sems[NBUF:2 * NBUF]
        msem = sems[2 * NBUF]
        dsem = sems[2 * NBUF + 1]
        c = lax.axis_index("c")
        s = lax.axis_index("s")
        cp_src = pltpu.make_async_copy(src_hbm.at[s], src_v, msem)
        cp_dst = pltpu.make_async_copy(dst_hbm.at[s], dst_v, msem)
        cp_zer = pltpu.make_async_copy(
            zer_hbm, agg_sh.at[pl.ds(s * RPT, RPT)], msem)
        cp_z16 = pltpu.make_async_copy(
            z16_hbm, deg_sh.at[pl.ds(s * RPT, RPT)], msem)
        cp_src.start()
        cp_dst.start()
        cp_zer.start()
        cp_z16.start()

        @pl.loop(0, SBLK // 2)
        def _(r):
            ones_v[pl.ds(r * 2, 2), :] = jnp.ones((2, 16), jnp.int16)

        cp_src.wait()
        cp_dst.wait()
        xc = x_hbm.at[c]
        for g in range(LOOK):
            pltpu.make_async_copy(
                xc.at[src_v.at[g]], bufs.at[g], gsem[g]).start()
        cp_zer.wait()
        cp_z16.wait()
        plsc.subcore_barrier()

        def extra(j):
            @pl.when(c == 0)
            def _():
                pltpu.make_async_copy(
                    ones_v, deg_sh.at[dst_v.at[j]], dsem).start(add=True)

            @pl.when(c == 1)
            def _():
                pltpu.make_async_copy(
                    ones_v, deg_sh.at[src_v.at[j]], dsem).start(add=True)

        _agg_loop(xc, src_v, dst_v, bufs, agg_sh, gsem, ssem, extra)

        @pl.loop(0, NSB)
        def _(j):
            pltpu.make_async_copy(ones_v, deg_sh.at[dst_v.at[0]], dsem).wait()

        plsc.subcore_barrier()
        pltpu.async_copy(
            agg_sh.at[pl.ds(s * RPT, RPT)],
            out_hbm.at[c].at[pl.ds(s * RPT, RPT)], msem).wait()

        @pl.when(c == 0)
        def _():
            pltpu.async_copy(
                deg_sh.at[pl.ds(s * RPT, RPT)],
                ind_hbm.at[pl.ds(s * RPT, RPT)], msem).wait()

        @pl.when(c == 1)
        def _():
            pltpu.async_copy(
                deg_sh.at[pl.ds(s * RPT, RPT)],
                outd_hbm.at[pl.ds(s * RPT, RPT)], msem).wait()

    return k


_AGG_DEG_SC = _make_agg_deg_sc()


def kernel(features, edge_index, noise, W0, W1, W2, W3, W4, W5, W6,
           b0, b1, b2, b3, b4, b5, b6):
    src_a = edge_index[0].reshape(16, NSB, SBLK)
    dst_a = edge_index[1].reshape(16, NSB, SBLK)

    zi16 = jnp.zeros((RPT, 16), jnp.int16)
    z32 = jnp.zeros((RPT, 32), jnp.float32)
    z64 = jnp.zeros((RPT, 64), jnp.float32)
    zb96 = jnp.zeros((RPT, 96), jnp.bfloat16)
    zb128 = jnp.zeros((RPT, 128), jnp.bfloat16)

    wcat = jnp.concatenate([W5, W6], axis=1)
    bcat = jnp.concatenate([b5, b6], axis=0)
    feat2 = jnp.stack([features[:, :64], features[:, 64:]])

    f32 = jnp.float32
    bf16 = jnp.bfloat16
    a0, ind, outd = _AGG_DEG_SC(feat2, src_a, dst_a, z64, zi16)
    xs1, rin, rout = _lin0(a0, W0, b0, ind, outd)
    a1 = _AGG64(xs1, src_a, dst_a, z64)
    (xs2,) = _linear((a1,), W1, b1, rin, rout, True, (64,), (f32,))
    a2 = _AGG64(xs2, src_a, dst_a, z64)
    (xs3,) = _linear((a2,), W2, b2, rin, rout, True, (96,), (bf16,))
    a3 = _AGG96B(xs3, src_a, dst_a, zb96)
    (xs4,) = _linear((a3,), W3, b3, rin, rout, True, (128,), (bf16,))
    a4 = _AGG128B(xs4, src_a, dst_a, zb128)
    y = _linear2(a4, W4, b4, wcat, rin, rout)
    a5 = _AGG32(y, src_a, dst_a, z32)
    mean, log_std, z = _head(a5, rin, bcat, noise)

    adj_rec = _decoder(z)
    return (adj_rec, mean, log_std)


# speedup vs baseline: 1.0979x; 1.0979x over previous
import functools

import jax
import jax.numpy as jnp
from jax import lax
from jax.experimental import pallas as pl
from jax.experimental.pallas import tpu as pltpu
from jax.experimental.pallas import tpu_sc as plsc

N = 10000
E = 160000
BN = 2000

SBLK = 200
NSB = 50
RPT = N // 16
NBUF = 5
LOOK = 4


def _lin0_body(x_ref, w_ref, b_ref, ind_ref, outd_ref, o_ref, rin_ref,
               rout_ref):
    ind = jnp.maximum(ind_ref[:, 0].astype(jnp.float32), 1.0)
    outd = jnp.maximum(outd_ref[:, 0].astype(jnp.float32), 1.0)
    inv = (1.0 / ind)[:, None]
    rin = jax.lax.rsqrt(ind)[:, None]
    rout = jax.lax.rsqrt(outd)[:, None]
    t = (jnp.dot(x_ref[0].astype(jnp.float32), w_ref[:64, :],
                 preferred_element_type=jnp.float32)
         + jnp.dot(x_ref[1].astype(jnp.float32), w_ref[64:, :],
                   preferred_element_type=jnp.float32))
    t = rout * jnp.maximum(inv * t + b_ref[...][None, :], 0.0)
    o_ref[0] = t[:, :64].astype(jnp.bfloat16)
    o_ref[1] = t[:, 64:].astype(jnp.bfloat16)
    rin_ref[...] = rin
    rout_ref[...] = rout


def _lin0(a0, w, b, ind, outd):
    dspec = pl.BlockSpec((BN, 16), lambda i: (i, 0))
    sspec = pl.BlockSpec((BN, 1), lambda i: (i, 0))
    return pl.pallas_call(
        _lin0_body,
        grid=(N // BN,),
        in_specs=[
            pl.BlockSpec((2, BN, 64), lambda i: (0, i, 0)),
            pl.BlockSpec((128, 128), lambda i: (0, 0)),
            pl.BlockSpec((128,), lambda i: (0,)),
            dspec,
            dspec,
        ],
        out_specs=[pl.BlockSpec((2, BN, 64), lambda i: (0, i, 0)),
                   sspec, sspec],
        out_shape=[jax.ShapeDtypeStruct((2, N, 64), jnp.bfloat16),
                   jax.ShapeDtypeStruct((N, 1), jnp.float32),
                   jax.ShapeDtypeStruct((N, 1), jnp.float32)],
    )(a0, w, b, ind, outd)


def _lin_body(ngroups, relu, post, *refs):
    x_refs = refs[:ngroups]
    w_ref, b_ref, spre_ref, spost_ref = refs[ngroups:ngroups + 4]
    o_refs = refs[ngroups + 4:]
    t = None
    off = 0
    for x_ref in x_refs:
        fh = x_ref.shape[2]
        for h in range(2):
            part = jnp.dot(x_ref[h].astype(jnp.float32),
                           w_ref[off:off + fh, :],
                           preferred_element_type=jnp.float32)
            t = part if t is None else t + part
            off += fh
    t = spre_ref[...] * t + b_ref[...][None, :]
    if relu:
        t = jnp.maximum(t, 0.0)
    if post:
        t = spost_ref[...] * t
    off = 0
    for o_ref in o_refs:
        hw = o_ref.shape[2]
        o_ref[0] = t[:, off:off + hw].astype(o_ref.dtype)
        o_ref[1] = t[:, off + hw:off + 2 * hw].astype(o_ref.dtype)
        off += 2 * hw


def _linear(x_groups, w, b, s_pre, s_post, relu, out_halves, out_dtypes):
    fo = w.shape[1]
    fi = w.shape[0]
    return pl.pallas_call(
        functools.partial(_lin_body, len(x_groups), relu, s_post is not None),
        grid=(N // BN,),
        in_specs=[pl.BlockSpec((2, BN, g.shape[2]), lambda i: (0, i, 0))
                  for g in x_groups] + [
            pl.BlockSpec((fi, fo), lambda i: (0, 0)),
            pl.BlockSpec((fo,), lambda i: (0,)),
            pl.BlockSpec((BN, 1), lambda i: (i, 0)),
            pl.BlockSpec((BN, 1), lambda i: (i, 0)),
        ],
        out_specs=[pl.BlockSpec((2, BN, hw), lambda i: (0, i, 0))
                   for hw in out_halves],
        out_shape=[jax.ShapeDtypeStruct((2, N, hw), dt)
                   for hw, dt in zip(out_halves, out_dtypes)],
    )(*x_groups, w, b, s_pre, s_pre if s_post is None else s_post)


def _lin2_body(x_ref, w_ref, b_ref, wc_ref, rin_ref, rout_ref, o_ref):
    t = (jnp.dot(x_ref[0].astype(jnp.float32), w_ref[:128, :],
                 preferred_element_type=jnp.float32)
         + jnp.dot(x_ref[1].astype(jnp.float32), w_ref[128:, :],
                   preferred_element_type=jnp.float32))
    t = jnp.maximum(rin_ref[...] * t + b_ref[...][None, :], 0.0)
    y = jnp.dot(rout_ref[...] * t, wc_ref[...],
                preferred_element_type=jnp.float32)
    o_ref[0] = y[:, :32]
    o_ref[1] = y[:, 32:]


def _linear2(x2, w, b, wcat, rin, rout):
    fo = w.shape[1]
    fc = wcat.shape[1]
    return pl.pallas_call(
        _lin2_body,
        grid=(N // BN,),
        in_specs=[
            pl.BlockSpec((2, BN, 128), lambda i: (0, i, 0)),
            pl.BlockSpec((256, fo), lambda i: (0, 0)),
            pl.BlockSpec((fo,), lambda i: (0,)),
            pl.BlockSpec((fo, fc), lambda i: (0, 0)),
            pl.BlockSpec((BN, 1), lambda i: (i, 0)),
            pl.BlockSpec((BN, 1), lambda i: (i, 0)),
        ],
        out_specs=pl.BlockSpec((2, BN, 32), lambda i: (0, i, 0)),
        out_shape=jax.ShapeDtypeStruct((2, N, 32), jnp.float32),
    )(x2, w, b, wcat, rin, rout)


def _head_body(x_ref, rin_ref, bc_ref, noise_ref, mean_ref, ls_ref, z_ref):
    mean = rin_ref[...] * x_ref[0].astype(jnp.float32) + bc_ref[...][None, :32]
    log_std = (rin_ref[...] * x_ref[1].astype(jnp.float32)
               + bc_ref[...][None, 32:])
    mean_ref[...] = mean
    ls_ref[...] = log_std
    z_ref[...] = mean + noise_ref[...] * jnp.exp(log_std * 0.5)


def _head(x2, rin, bcat, noise):
    o32 = pl.BlockSpec((BN, 32), lambda i: (i, 0))
    return pl.pallas_call(
        _head_body,
        grid=(N // BN,),
        in_specs=[
            pl.BlockSpec((2, BN, 32), lambda i: (0, i, 0)),
            pl.BlockSpec((BN, 1), lambda i: (i, 0)),
            pl.BlockSpec((64,), lambda i: (0,)),
            o32,
        ],
        out_specs=[o32, o32, o32],
        out_shape=[jax.ShapeDtypeStruct((N, 32), jnp.float32)] * 3,
    )(x2, rin, bcat, noise)


def _dec_body(za_ref, zb_ref, o_ref):
    t = jnp.dot(za_ref[...], zb_ref[...].T, preferred_element_type=jnp.float32)
    o_ref[...] = jax.nn.sigmoid(t)


def _decoder(z):
    br, bc = 1000, 1280
    return pl.pallas_call(
        _dec_body,
        grid=(N // br, pl.cdiv(N, bc)),
        in_specs=[
            pl.BlockSpec((br, 32), lambda i, j: (i, 0)),
            pl.BlockSpec((bc, 32), lambda i, j: (j, 0)),
        ],
        out_specs=pl.BlockSpec((br, bc), lambda i, j: (i, j)),
        out_shape=jax.ShapeDtypeStruct((N, N), jnp.float32),
    )(z, z)


def _agg_loop(xc, src_v, dst_v, bufs, agg_sh, gsem, ssem, extra=None):
    @pl.loop(0, NSB, step=NBUF)
    def _(j0):
        for u in range(NBUF):
            j = j0 + u
            if extra is not None:
                extra(j)
            pltpu.make_async_copy(
                xc.at[src_v.at[j]], bufs.at[u], gsem[u]).wait()
            pltpu.make_async_copy(
                bufs.at[u], agg_sh.at[dst_v.at[j]], ssem[u]).start(add=True)
            g2 = j + LOOK
            u2 = (u + LOOK) % NBUF

            @pl.when(g2 < NSB)
            def _():
                @pl.when(g2 >= NBUF)
                def _():
                    pltpu.make_async_copy(
                        bufs.at[u2], agg_sh.at[dst_v.at[g2 - NBUF]],
                        ssem[u2]).wait()

                pltpu.make_async_copy(
                    xc.at[src_v.at[g2]], bufs.at[u2], gsem[u2]).start()

    for u in range(NBUF):
        pltpu.make_async_copy(
            bufs.at[u], agg_sh.at[dst_v.at[NSB - NBUF + u]], ssem[u]).wait()


def _make_agg_sc(FH, dtype):
    mesh = plsc.VectorSubcoreMesh(core_axis_name="c", subcore_axis_name="s")
    scratch = [
        pltpu.VMEM((NSB, SBLK), jnp.int32),
        pltpu.VMEM((NSB, SBLK), jnp.int32),
        pltpu.VMEM((NBUF, SBLK, FH), dtype),
        pltpu.VMEM_SHARED((N, FH), dtype),
    ] + [pltpu.SemaphoreType.DMA] * (2 * NBUF + 1)

    @functools.partial(
        pl.kernel,
        out_type=jax.ShapeDtypeStruct((2, N, FH), dtype),
        mesh=mesh,
        scratch_types=scratch,
        compiler_params=pltpu.CompilerParams(use_tc_tiling_on_sc=False),
    )
    def k(x_hbm, src_hbm, dst_hbm, zer_hbm, out_hbm, src_v, dst_v, bufs,
          agg_sh, *sems):
        gsem = sems[:NBUF]
        ssem = sems[NBUF:2 * NBUF]
        msem = sems[2 * NBUF]
        c = lax.axis_index("c")
        s = lax.axis_index("s")
        cp_src = pltpu.make_async_copy(src_hbm.at[s], src_v, msem)
        cp_dst = pltpu.make_async_copy(dst_hbm.at[s], dst_v, msem)
        cp_zer = pltpu.make_async_copy(
            zer_hbm, agg_sh.at[pl.ds(s * RPT, RPT)], msem)
        cp_src.start()
        cp_dst.start()
        cp_zer.start()
        cp_src.wait()
        cp_dst.wait()
        xc = x_hbm.at[c]
        for g in range(LOOK):
            pltpu.make_async_copy(
                xc.at[src_v.at[g]], bufs.at[g], gsem[g]).start()
        cp_zer.wait()
        plsc.subcore_barrier()
        _agg_loop(xc, src_v, dst_v, bufs, agg_sh, gsem, ssem)
        plsc.subcore_barrier()
        pltpu.async_copy(
            agg_sh.at[pl.ds(s * RPT, RPT)],
            out_hbm.at[c].at[pl.ds(s * RPT, RPT)], msem).wait()

    return k


_AGG32 = _make_agg_sc(32, jnp.float32)
_AGG64 = _make_agg_sc(64, jnp.bfloat16)
_AGG96B = _make_agg_sc(96, jnp.bfloat16)
_AGG128B = _make_agg_sc(128, jnp.bfloat16)


def _make_agg_deg_sc():
    FH = 64
    mesh = plsc.VectorSubcoreMesh(core_axis_name="c", subcore_axis_name="s")
    scratch = [
        pltpu.VMEM((NSB, SBLK), jnp.int32),
        pltpu.VMEM((NSB, SBLK), jnp.int32),
        pltpu.VMEM((NBUF, SBLK, FH), jnp.float32),
        pltpu.VMEM((SBLK, 16), jnp.int16),
        pltpu.VMEM_SHARED((N, FH), jnp.float32),
        pltpu.VMEM_SHARED((N, 16), jnp.int16),
    ] + [pltpu.SemaphoreType.DMA] * (2 * NBUF + 2)

    @functools.partial(
        pl.kernel,
        out_type=[jax.ShapeDtypeStruct((2, N, FH), jnp.float32),
                  jax.ShapeDtypeStruct((N, 16), jnp.int16),
                  jax.ShapeDtypeStruct((N, 16), jnp.int16)],
        mesh=mesh,
        scratch_types=scratch,
        compiler_params=pltpu.CompilerParams(use_tc_tiling_on_sc=False),
    )
    def k(x_hbm, src_hbm, dst_hbm, zer_hbm, z16_hbm, out_hbm, ind_hbm,
          outd_hbm, src_v, dst_v, bufs, ones_v, agg_sh, deg_sh, *sems):
        gsem = sems[:NBUF]
        ssem = sems[NBUF:2 * NBUF]
        msem = sems[2 * NBUF]
        dsem = sems[2 * NBUF + 1]
        c = lax.axis_index("c")
        s = lax.axis_index("s")
        cp_src = pltpu.make_async_copy(src_hbm.at[s], src_v, msem)
        cp_dst = pltpu.make_async_copy(dst_hbm.at[s], dst_v, msem)
        cp_zer = pltpu.make_async_copy(
            zer_hbm, agg_sh.at[pl.ds(s * RPT, RPT)], msem)
        cp_z16 = pltpu.make_async_copy(
            z16_hbm, deg_sh.at[pl.ds(s * RPT, RPT)], msem)
        cp_src.start()
        cp_dst.start()
        cp_zer.start()
        cp_z16.start()

        @pl.loop(0, SBLK // 2)
        def _(r):
            ones_v[pl.ds(r * 2, 2), :] = jnp.ones((2, 16), jnp.int16)

        cp_src.wait()
        cp_dst.wait()
        xc = x_hbm.at[c]
        for g in range(LOOK):
            pltpu.make_async_copy(
                xc.at[src_v.at[g]], bufs.at[g], gsem[g]).start()
        cp_zer.wait()
        cp_z16.wait()
        plsc.subcore_barrier()

        def extra(j):
            @pl.when(c == 0)
            def _():
                pltpu.make_async_copy(
                    ones_v, deg_sh.at[dst_v.at[j]], dsem).start(add=True)

            @pl.when(c == 1)
            def _():
                pltpu.make_async_copy(
                    ones_v, deg_sh.at[src_v.at[j]], dsem).start(add=True)

        _agg_loop(xc, src_v, dst_v, bufs, agg_sh, gsem, ssem, extra)

        @pl.loop(0, NSB)
        def _(j):
            pltpu.make_async_copy(ones_v, deg_sh.at[dst_v.at[0]], dsem).wait()

        plsc.subcore_barrier()
        pltpu.async_copy(
            agg_sh.at[pl.ds(s * RPT, RPT)],
            out_hbm.at[c].at[pl.ds(s * RPT, RPT)], msem).wait()

        @pl.when(c == 0)
        def _():
            pltpu.async_copy(
                deg_sh.at[pl.ds(s * RPT, RPT)],
                ind_hbm.at[pl.ds(s * RPT, RPT)], msem).wait()

        @pl.when(c == 1)
        def _():
            pltpu.async_copy(
                deg_sh.at[pl.ds(s * RPT, RPT)],
                outd_hbm.at[pl.ds(s * RPT, RPT)], msem).wait()

    return k


_AGG_DEG_SC = _make_agg_deg_sc()


def kernel(features, edge_index, noise, W0, W1, W2, W3, W4, W5, W6,
           b0, b1, b2, b3, b4, b5, b6):
    src_a = edge_index[0].reshape(16, NSB, SBLK)
    dst_a = edge_index[1].reshape(16, NSB, SBLK)

    zi16 = jnp.zeros((RPT, 16), jnp.int16)
    zf64 = jnp.zeros((RPT, 64), jnp.float32)
    z32 = jnp.zeros((RPT, 32), jnp.float32)
    z64 = jnp.zeros((RPT, 64), jnp.bfloat16)
    zb96 = jnp.zeros((RPT, 96), jnp.bfloat16)
    zb128 = jnp.zeros((RPT, 128), jnp.bfloat16)

    wcat = jnp.concatenate([W5, W6], axis=1)
    bcat = jnp.concatenate([b5, b6], axis=0)
    feat2 = jnp.stack([features[:, :64], features[:, 64:]])

    f32 = jnp.float32
    bf16 = jnp.bfloat16
    a0, ind, outd = _AGG_DEG_SC(feat2, src_a, dst_a, zf64, zi16)
    xs1, rin, rout = _lin0(a0, W0, b0, ind, outd)
    a1 = _AGG64(xs1, src_a, dst_a, z64)
    (xs2,) = _linear((a1,), W1, b1, rin, rout, True, (64,), (bf16,))
    a2 = _AGG64(xs2, src_a, dst_a, z64)
    (xs3,) = _linear((a2,), W2, b2, rin, rout, True, (96,), (bf16,))
    a3 = _AGG96B(xs3, src_a, dst_a, zb96)
    (xs4,) = _linear((a3,), W3, b3, rin, rout, True, (128,), (bf16,))
    a4 = _AGG128B(xs4, src_a, dst_a, zb128)
    y = _linear2(a4, W4, b4, wcat, rin, rout)
    a5 = _AGG32(y, src_a, dst_a, z32)
    mean, log_std, z = _head(a5, rin, bcat, noise)

    adj_rec = _decoder(z)
    return (adj_rec, mean, log_std)
